# Initial kernel scaffold; baseline (speedup 1.0000x reference)
import jax
import jax.numpy as jnp
from jax.experimental import pallas as pl


def kernel(x, edge_index, W1_l, W1_r, b1, W2_l, W2_r, b2):
    raise NotImplementedError("write your pallas kernel here")


# speedup vs baseline: 7.6732x; 7.6732x over previous
import functools

import jax
import jax.numpy as jnp
from jax import lax
from jax.experimental import pallas as pl
from jax.experimental.pallas import tpu as pltpu
from jax.experimental.pallas import tpu_sc as plsc

_N = 10000
_E = 320000
_D = 128
_C = 47
_C_PAD = 48

_NC = 2
_NS = 16
_NW = _NC * _NS

_CH = 128
_NCHUNK = 79
_EPW = _NCHUNK * _CH
_EPAD = _NW * _EPW

_NROWS = 10240
_RPS = _NROWS // _NS
_ZCH = 128

_mesh = plsc.VectorSubcoreMesh(core_axis_name="c", subcore_axis_name="s")


def _zero_vmem(ref, rows, width):
    def body(i, carry):
        for j in range(width // 16):
            ref[i, pl.ds(j * 16, 16)] = jnp.zeros((16,), jnp.float32)
        return carry
    lax.fori_loop(0, rows, body, 0)


@functools.partial(
    pl.kernel,
    mesh=_mesh,
    out_type=[
        jax.ShapeDtypeStruct((_NC, _NROWS, _D), jnp.float32),
    ],
    scratch_types=[
        pltpu.VMEM((_NCHUNK, _CH), jnp.int32),
        pltpu.VMEM((_CH, _D), jnp.float32),
        pltpu.VMEM((_CH, _D), jnp.float32),
        pltpu.VMEM_SHARED((_NROWS, _D), jnp.float32),
        pltpu.SemaphoreType.DMA,
    ],
)
def _sc_pass0(dst_hbm, out_cnt, dst_v, ones_v, z16_v, cnt_s, sem):
    c = lax.axis_index("c")
    s = lax.axis_index("s")
    wid = s * _NC + c

    _zero_vmem(z16_v, _CH, _D)

    def fill_ones(i, carry):
        for j in range(_D // 16):
            ones_v[i, pl.ds(j * 16, 16)] = jnp.ones((16,), jnp.float32)
        return carry
    lax.fori_loop(0, _CH, fill_ones, 0)

    for k in range(_RPS // _ZCH):
        pltpu.sync_copy(z16_v, cnt_s.at[pl.ds(s * _RPS + k * _ZCH, _ZCH)])
    plsc.subcore_barrier()

    pltpu.sync_copy(dst_hbm.at[wid], dst_v)

    def chunk(i, carry):
        pltpu.sync_copy(ones_v, cnt_s.at[dst_v.at[i]], add=True)
        return carry
    lax.fori_loop(0, _NCHUNK, chunk, 0)

    plsc.subcore_barrier()
    rbase = s * _RPS
    pltpu.sync_copy(cnt_s.at[pl.ds(rbase, _RPS)],
                    out_cnt.at[c].at[pl.ds(rbase, _RPS)])


def _make_sum_pass():
    @functools.partial(
        pl.kernel,
        mesh=_mesh,
        out_type=[
            jax.ShapeDtypeStruct((_NC, _NROWS, _D), jnp.float32),
        ],
        scratch_types=[
            pltpu.VMEM((_NCHUNK, _CH), jnp.int32),
            pltpu.VMEM((_NCHUNK, _CH), jnp.int32),
            pltpu.VMEM((_CH, _D), jnp.float32),
            pltpu.VMEM_SHARED((_NROWS, _D), jnp.float32),
            pltpu.SemaphoreType.DMA,
        ],
    )
    def sum_pass(tab_hbm, src_hbm, dst_hbm, out_sum,
                 src_v, dst_v, rows_v, acc_s, sem):
        c = lax.axis_index("c")
        s = lax.axis_index("s")
        wid = s * _NC + c

        _zero_vmem(rows_v, _CH, _D)
        for k in range(_RPS // _ZCH):
            pltpu.sync_copy(rows_v, acc_s.at[pl.ds(s * _RPS + k * _ZCH, _ZCH)])
        plsc.subcore_barrier()

        pltpu.sync_copy(src_hbm.at[wid], src_v)
        pltpu.sync_copy(dst_hbm.at[wid], dst_v)

        def chunk(i, carry):
            pltpu.async_copy(tab_hbm.at[src_v.at[i]], rows_v, sem).wait()
            pltpu.sync_copy(rows_v, acc_s.at[dst_v.at[i]], add=True)
            return carry
        lax.fori_loop(0, _NCHUNK, chunk, 0)

        plsc.subcore_barrier()
        rbase = s * _RPS
        pltpu.sync_copy(acc_s.at[pl.ds(rbase, _RPS)],
                        out_sum.at[c].at[pl.ds(rbase, _RPS)])
    return sum_pass


_sc_pass1 = _make_sum_pass()
_sc_pass2 = _make_sum_pass()

_R = 1000


def _tc1_body(p0, p1, c0, c1, x, w1l, w1r, b1, w2r, b2, h_out, r_out):
    cnt = jnp.maximum(c0[:, 0:1] + c1[:, 0:1], 1.0)
    mean = (p0[...] + p1[...]) / cnt
    h = jnp.dot(mean, w1l[...], preferred_element_type=jnp.float32)
    h = h + jnp.dot(x[...], w1r[...], preferred_element_type=jnp.float32)
    h = jnp.maximum(h + b1[...], 0.0)
    h_out[...] = h
    r_out[...] = jnp.dot(h, w2r[...], preferred_element_type=jnp.float32) + b2[...]


def _tc2_body(q0, q1, c0, c1, r, w2l, o_out):
    cnt = jnp.maximum(c0[:, 0:1] + c1[:, 0:1], 1.0)
    mean2 = (q0[...] + q1[...]) / cnt
    z = jnp.dot(mean2, w2l[...], preferred_element_type=jnp.float32) + r[...]
    col = lax.broadcasted_iota(jnp.int32, (_R, _C_PAD), 1)
    valid = col < _C
    zm = jnp.where(valid, z, -1e30)
    m = jnp.max(zm, axis=1, keepdims=True)
    e = jnp.where(valid, jnp.exp(z - m), 0.0)
    lse = jnp.log(jnp.sum(e, axis=1, keepdims=True))
    o_out[...] = z - m - lse


def _row_spec(w):
    return pl.BlockSpec((_R, w), lambda i: (i, 0))


def _full_spec(shape):
    return pl.BlockSpec(shape, lambda i: (0,) * len(shape))


_tc1 = pl.pallas_call(
    _tc1_body,
    grid=(_N // _R,),
    in_specs=[
        _row_spec(_D), _row_spec(_D), _row_spec(_D), _row_spec(_D),
        _row_spec(_D),
        _full_spec((_D, _D)), _full_spec((_D, _D)), _full_spec((1, _D)),
        _full_spec((_D, _C_PAD)),
        _full_spec((1, _C_PAD)),
    ],
    out_specs=[_row_spec(_D), _row_spec(_C_PAD)],
    out_shape=[
        jax.ShapeDtypeStruct((_N, _D), jnp.float32),
        jax.ShapeDtypeStruct((_N, _C_PAD), jnp.float32),
    ],
)

_tc2 = pl.pallas_call(
    _tc2_body,
    grid=(_N // _R,),
    in_specs=[
        _row_spec(_D), _row_spec(_D), _row_spec(_D), _row_spec(_D),
        _row_spec(_C_PAD), _full_spec((_D, _C_PAD)),
    ],
    out_specs=[_row_spec(_C_PAD)],
    out_shape=[jax.ShapeDtypeStruct((_N, _C_PAD), jnp.float32)],
)


def kernel(x, edge_index, W1_l, W1_r, b1, W2_l, W2_r, b2):
    src = edge_index[0].astype(jnp.int32)
    dst = edge_index[1].astype(jnp.int32)

    npad = _EPAD - _E
    ar = jnp.arange(npad, dtype=jnp.int32)
    pad_src = ar % _N
    pad_dst = _N + ar % (_NROWS - _N)
    src_p = jnp.concatenate([src, pad_src]).reshape(_NW, _NCHUNK, _CH)
    dst_p = jnp.concatenate([dst, pad_dst]).reshape(_NW, _NCHUNK, _CH)

    (cnt1,) = _sc_pass0(dst_p)
    (sum1,) = _sc_pass1(x, src_p, dst_p)
    p0 = sum1[0, :_N]
    p1 = sum1[1, :_N]
    c0 = cnt1[0, :_N]
    c1 = cnt1[1, :_N]

    w2l = jnp.pad(W2_l, ((0, 0), (0, _C_PAD - _C)))
    w2r = jnp.pad(W2_r, ((0, 0), (0, _C_PAD - _C)))
    b2p = jnp.pad(b2, (0, _C_PAD - _C)).reshape(1, _C_PAD)

    h, r = _tc1(p0, p1, c0, c1, x, W1_l, W1_r, b1.reshape(1, _D), w2r, b2p)

    (sum2,) = _sc_pass2(h, src_p, dst_p)
    q0 = sum2[0, :_N]
    q1 = sum2[1, :_N]

    (o,) = _tc2(q0, q1, c0, c1, r, w2l)
    return o[:, :_C]


# speedup vs baseline: 9.3111x; 1.2135x over previous
import functools

import jax
import jax.numpy as jnp
from jax import lax
from jax.experimental import pallas as pl
from jax.experimental.pallas import tpu as pltpu
from jax.experimental.pallas import tpu_sc as plsc

_N = 10000
_E = 320000
_D = 128
_C = 47
_C_PAD = 48

_NC = 2
_NS = 16
_NW = _NC * _NS

_CH = 128
_NCHUNK = 80
_EPW = _NCHUNK * _CH
_EPAD = _NW * _EPW
_NBUF = 2
_IH = _NCHUNK // 2

_NROWS = 10240
_RPS = _NROWS // _NS
_ZCH = 128

_mesh = plsc.VectorSubcoreMesh(core_axis_name="c", subcore_axis_name="s")


def _zero_vmem(ref, rows, width):
    def body(i, carry):
        for j in range(width // 16):
            ref[i, pl.ds(j * 16, 16)] = jnp.zeros((16,), jnp.float32)
        return carry
    lax.fori_loop(0, rows, body, 0)


@functools.partial(
    pl.kernel,
    mesh=_mesh,
    out_type=[
        jax.ShapeDtypeStruct((_NC, _NROWS, _D), jnp.float32),
    ],
    scratch_types=[
        pltpu.VMEM((_NCHUNK, _CH), jnp.int32),
        pltpu.VMEM((_CH, _D), jnp.float32),
        pltpu.VMEM_SHARED((_NROWS, _D), jnp.float32),
        pltpu.SemaphoreType.DMA,
    ],
)
def _sc_pass0(dst_hbm, out_cnt, dst_v, ones_v, cnt_s, sem):
    c = lax.axis_index("c")
    s = lax.axis_index("s")
    wid = s * _NC + c

    _zero_vmem(ones_v, _CH, _D)
    for k in range(_RPS // _ZCH):
        pltpu.sync_copy(ones_v, cnt_s.at[pl.ds(s * _RPS + k * _ZCH, _ZCH)])

    def fill_ones(i, carry):
        for j in range(_D // 16):
            ones_v[i, pl.ds(j * 16, 16)] = jnp.ones((16,), jnp.float32)
        return carry
    lax.fori_loop(0, _CH, fill_ones, 0)
    plsc.subcore_barrier()

    pltpu.sync_copy(dst_hbm.at[wid], dst_v)

    def chunk(j, carry):
        for b in range(8):
            pltpu.async_copy(ones_v, cnt_s.at[dst_v.at[j * 8 + b]], sem,
                             add=True)
        for b in range(8):
            pltpu.make_async_copy(ones_v, cnt_s.at[dst_v.at[j * 8 + b]],
                                  sem).wait()
        return carry
    lax.fori_loop(0, _NCHUNK // 8, chunk, 0)

    plsc.subcore_barrier()
    rbase = s * _RPS
    pltpu.sync_copy(cnt_s.at[pl.ds(rbase, _RPS)],
                    out_cnt.at[c].at[pl.ds(rbase, _RPS)])


def _make_sum_pass():
    @functools.partial(
        pl.kernel,
        mesh=_mesh,
        out_type=[
            jax.ShapeDtypeStruct((_NC, _NROWS, _D), jnp.float32),
        ],
        scratch_types=[
            pltpu.VMEM((_IH, _CH), jnp.int32),
            pltpu.VMEM((_IH, _CH), jnp.int32),
            pltpu.VMEM((_NBUF, _CH, _D), jnp.float32),
            pltpu.VMEM_SHARED((_NROWS, _D), jnp.float32),
            [pltpu.SemaphoreType.DMA] * _NBUF,
        ],
    )
    def sum_pass(tab_hbm, src_hbm, dst_hbm, out_sum,
                 src_v, dst_v, rows_v, acc_s, gsem):
        c = lax.axis_index("c")
        s = lax.axis_index("s")
        wid = s * _NC + c

        _zero_vmem(rows_v.at[0], _CH, _D)
        for k in range(_RPS // _ZCH):
            pltpu.sync_copy(rows_v.at[0], acc_s.at[pl.ds(s * _RPS + k * _ZCH, _ZCH)])
        plsc.subcore_barrier()

        nj = _IH // _NBUF
        for h in range(2):
            pltpu.sync_copy(src_hbm.at[wid].at[pl.ds(h * _IH, _IH)], src_v)
            pltpu.sync_copy(dst_hbm.at[wid].at[pl.ds(h * _IH, _IH)], dst_v)
            pltpu.async_copy(tab_hbm.at[src_v.at[0]], rows_v.at[0], gsem[0])

            def group(j, carry):
                for b in range(_NBUF):
                    i = j * _NBUF + b
                    bn = (b + 1) % _NBUF
                    pltpu.make_async_copy(tab_hbm.at[src_v.at[i]],
                                          rows_v.at[b], gsem[b]).wait()
                    if b == 0:
                        pltpu.async_copy(tab_hbm.at[src_v.at[i + 1]],
                                         rows_v.at[bn], gsem[bn])
                    else:
                        @pl.when(j < nj - 1)
                        def _fire():
                            pltpu.async_copy(tab_hbm.at[src_v.at[i + 1]],
                                             rows_v.at[bn], gsem[bn])
                    pltpu.sync_copy(rows_v.at[b], acc_s.at[dst_v.at[i]],
                                    add=True)
                return carry
            lax.fori_loop(0, nj, group, 0)

        plsc.subcore_barrier()
        rbase = s * _RPS
        pltpu.sync_copy(acc_s.at[pl.ds(rbase, _RPS)],
                        out_sum.at[c].at[pl.ds(rbase, _RPS)])
    return sum_pass


_sc_pass1 = _make_sum_pass()
_sc_pass2 = _make_sum_pass()

_R = 1000


def _tc1_body(p0, p1, c0, c1, x, w1l, w1r, b1, w2r, b2, h_out, r_out):
    cnt = jnp.maximum(c0[:, 0:1] + c1[:, 0:1], 1.0)
    mean = (p0[...] + p1[...]) / cnt
    h = jnp.dot(mean, w1l[...], preferred_element_type=jnp.float32)
    h = h + jnp.dot(x[...], w1r[...], preferred_element_type=jnp.float32)
    h = jnp.maximum(h + b1[...], 0.0)
    h_out[...] = h
    r_out[...] = jnp.dot(h, w2r[...], preferred_element_type=jnp.float32) + b2[...]


def _tc2_body(q0, q1, c0, c1, r, w2l, o_out):
    cnt = jnp.maximum(c0[:, 0:1] + c1[:, 0:1], 1.0)
    mean2 = (q0[...] + q1[...]) / cnt
    z = jnp.dot(mean2, w2l[...], preferred_element_type=jnp.float32) + r[...]
    col = lax.broadcasted_iota(jnp.int32, (_R, _C_PAD), 1)
    valid = col < _C
    zm = jnp.where(valid, z, -1e30)
    m = jnp.max(zm, axis=1, keepdims=True)
    e = jnp.where(valid, jnp.exp(z - m), 0.0)
    lse = jnp.log(jnp.sum(e, axis=1, keepdims=True))
    o_out[...] = z - m - lse


def _row_spec(w):
    return pl.BlockSpec((_R, w), lambda i: (i, 0))


def _full_spec(shape):
    return pl.BlockSpec(shape, lambda i: (0,) * len(shape))


_tc1 = pl.pallas_call(
    _tc1_body,
    grid=(_N // _R,),
    in_specs=[
        _row_spec(_D), _row_spec(_D), _row_spec(_D), _row_spec(_D),
        _row_spec(_D),
        _full_spec((_D, _D)), _full_spec((_D, _D)), _full_spec((1, _D)),
        _full_spec((_D, _C_PAD)),
        _full_spec((1, _C_PAD)),
    ],
    out_specs=[_row_spec(_D), _row_spec(_C_PAD)],
    out_shape=[
        jax.ShapeDtypeStruct((_N, _D), jnp.float32),
        jax.ShapeDtypeStruct((_N, _C_PAD), jnp.float32),
    ],
)

_tc2 = pl.pallas_call(
    _tc2_body,
    grid=(_N // _R,),
    in_specs=[
        _row_spec(_D), _row_spec(_D), _row_spec(_D), _row_spec(_D),
        _row_spec(_C_PAD), _full_spec((_D, _C_PAD)),
    ],
    out_specs=[_row_spec(_C_PAD)],
    out_shape=[jax.ShapeDtypeStruct((_N, _C_PAD), jnp.float32)],
)


def kernel(x, edge_index, W1_l, W1_r, b1, W2_l, W2_r, b2):
    src = edge_index[0].astype(jnp.int32)
    dst = edge_index[1].astype(jnp.int32)

    npad = _EPAD - _E
    ar = jnp.arange(npad, dtype=jnp.int32)
    pad_src = ar % _N
    pad_dst = _N + ar % (_NROWS - _N)
    src_p = jnp.concatenate([src, pad_src]).reshape(_NW, _NCHUNK, _CH)
    dst_p = jnp.concatenate([dst, pad_dst]).reshape(_NW, _NCHUNK, _CH)

    (cnt1,) = _sc_pass0(dst_p)
    (sum1,) = _sc_pass1(x, src_p, dst_p)
    p0 = sum1[0, :_N]
    p1 = sum1[1, :_N]
    c0 = cnt1[0, :_N]
    c1 = cnt1[1, :_N]

    w2l = jnp.pad(W2_l, ((0, 0), (0, _C_PAD - _C)))
    w2r = jnp.pad(W2_r, ((0, 0), (0, _C_PAD - _C)))
    b2p = jnp.pad(b2, (0, _C_PAD - _C)).reshape(1, _C_PAD)

    h, r = _tc1(p0, p1, c0, c1, x, W1_l, W1_r, b1.reshape(1, _D), w2r, b2p)

    (sum2,) = _sc_pass2(h, src_p, dst_p)
    q0 = sum2[0, :_N]
    q1 = sum2[1, :_N]

    (o,) = _tc2(q0, q1, c0, c1, r, w2l)
    return o[:, :_C]


# speedup vs baseline: 11.6454x; 1.2507x over previous
import functools

import jax
import jax.numpy as jnp
from jax import lax
from jax.experimental import pallas as pl
from jax.experimental.pallas import tpu as pltpu
from jax.experimental.pallas import tpu_sc as plsc

_N = 10000
_E = 320000
_D = 128
_C = 47
_C_PAD = 48

_NC = 2
_NS = 16
_NW = _NC * _NS

_CH = 128
_NCHUNK = 80
_EPW = _NCHUNK * _CH
_EPAD = _NW * _EPW
_NBUF = 2
_IH = _NCHUNK // 2

_NROWS = 10240
_RPS = _NROWS // _NS
_ZCH = 128

_mesh = plsc.VectorSubcoreMesh(core_axis_name="c", subcore_axis_name="s")
_sc_params = pltpu.CompilerParams(use_tc_tiling_on_sc=False)


def _zero_vmem(ref, rows, width):
    def body(i, carry):
        for j in range(width // 16):
            ref[i, pl.ds(j * 16, 16)] = jnp.zeros((16,), jnp.float32)
        return carry
    lax.fori_loop(0, rows, body, 0)


@functools.partial(
    pl.kernel,
    mesh=_mesh,
    out_type=[
        jax.ShapeDtypeStruct((_NC, _NROWS, 16), jnp.float32),
    ],
    scratch_types=[
        pltpu.VMEM((_NCHUNK, _CH), jnp.int32),
        pltpu.VMEM((_CH, 16), jnp.float32),
        pltpu.VMEM_SHARED((_NROWS, 16), jnp.float32),
        pltpu.SemaphoreType.DMA,
    ],
    compiler_params=_sc_params,
)
def _sc_pass0(dst_hbm, out_cnt, dst_v, ones_v, cnt_s, sem):
    c = lax.axis_index("c")
    s = lax.axis_index("s")
    wid = s * _NC + c

    _zero_vmem(ones_v, _CH, 16)
    for k in range(_RPS // _ZCH):
        pltpu.sync_copy(ones_v, cnt_s.at[pl.ds(s * _RPS + k * _ZCH, _ZCH)])

    def fill_ones(i, carry):
        ones_v[i, pl.ds(0, 16)] = jnp.ones((16,), jnp.float32)
        return carry
    lax.fori_loop(0, _CH, fill_ones, 0)
    plsc.subcore_barrier()

    pltpu.sync_copy(dst_hbm.at[wid], dst_v)

    def chunk(j, carry):
        for b in range(8):
            pltpu.async_copy(ones_v, cnt_s.at[dst_v.at[j * 8 + b]], sem,
                             add=True)
        for b in range(8):
            pltpu.make_async_copy(ones_v, cnt_s.at[dst_v.at[j * 8 + b]],
                                  sem).wait()
        return carry
    lax.fori_loop(0, _NCHUNK // 8, chunk, 0)

    plsc.subcore_barrier()
    rbase = s * _RPS
    pltpu.sync_copy(cnt_s.at[pl.ds(rbase, _RPS)],
                    out_cnt.at[c].at[pl.ds(rbase, _RPS)])


def _make_sum_pass(width):
    @functools.partial(
        pl.kernel,
        mesh=_mesh,
        out_type=[
            jax.ShapeDtypeStruct((_NC, _NROWS, width), jnp.float32),
        ],
        scratch_types=[
            pltpu.VMEM((_IH, _CH), jnp.int32),
            pltpu.VMEM((_IH, _CH), jnp.int32),
            pltpu.VMEM((_NBUF, _CH, width), jnp.float32),
            pltpu.VMEM_SHARED((_NROWS, width), jnp.float32),
            [pltpu.SemaphoreType.DMA] * _NBUF,
        ],
        compiler_params=_sc_params,
    )
    def sum_pass(tab_hbm, src_hbm, dst_hbm, out_sum,
                 src_v, dst_v, rows_v, acc_s, gsem):
        c = lax.axis_index("c")
        s = lax.axis_index("s")
        wid = s * _NC + c

        _zero_vmem(rows_v.at[0], _CH, width)
        for k in range(_RPS // _ZCH):
            pltpu.sync_copy(rows_v.at[0], acc_s.at[pl.ds(s * _RPS + k * _ZCH, _ZCH)])
        plsc.subcore_barrier()

        nj = _IH // _NBUF
        for h in range(2):
            pltpu.sync_copy(src_hbm.at[wid].at[pl.ds(h * _IH, _IH)], src_v)
            pltpu.sync_copy(dst_hbm.at[wid].at[pl.ds(h * _IH, _IH)], dst_v)
            pltpu.async_copy(tab_hbm.at[src_v.at[0]], rows_v.at[0], gsem[0])

            def group(j, carry):
                for b in range(_NBUF):
                    i = j * _NBUF + b
                    bn = (b + 1) % _NBUF
                    pltpu.make_async_copy(tab_hbm.at[src_v.at[i]],
                                          rows_v.at[b], gsem[b]).wait()
                    if b == 0:
                        pltpu.async_copy(tab_hbm.at[src_v.at[i + 1]],
                                         rows_v.at[bn], gsem[bn])
                    else:
                        @pl.when(j < nj - 1)
                        def _fire():
                            pltpu.async_copy(tab_hbm.at[src_v.at[i + 1]],
                                             rows_v.at[bn], gsem[bn])
                    pltpu.sync_copy(rows_v.at[b], acc_s.at[dst_v.at[i]],
                                    add=True)
                return carry
            lax.fori_loop(0, nj, group, 0)

        plsc.subcore_barrier()
        rbase = s * _RPS
        pltpu.sync_copy(acc_s.at[pl.ds(rbase, _RPS)],
                        out_sum.at[c].at[pl.ds(rbase, _RPS)])
    return sum_pass


_sc_pass1 = _make_sum_pass(_D)
_sc_pass2 = _make_sum_pass(_C_PAD)

_R = 1000


def _tc1_body(p0, p1, c0, c1, x, w1l, w1r, b1, w2l, w2r, b2, g_out, r_out):
    cnt = jnp.maximum(c0[:, 0:1] + c1[:, 0:1], 1.0)
    mean = (p0[...] + p1[...]) / cnt
    h = jnp.dot(mean, w1l[...], preferred_element_type=jnp.float32)
    h = h + jnp.dot(x[...], w1r[...], preferred_element_type=jnp.float32)
    h = jnp.maximum(h + b1[...], 0.0)
    g_out[...] = jnp.dot(h, w2l[...], preferred_element_type=jnp.float32)
    r_out[...] = jnp.dot(h, w2r[...], preferred_element_type=jnp.float32) + b2[...]


def _tc2_body(q0, q1, c0, c1, r, o_out):
    cnt = jnp.maximum(c0[:, 0:1] + c1[:, 0:1], 1.0)
    z = (q0[...] + q1[...]) / cnt + r[...]
    col = lax.broadcasted_iota(jnp.int32, (_R, _C_PAD), 1)
    valid = col < _C
    zm = jnp.where(valid, z, -1e30)
    m = jnp.max(zm, axis=1, keepdims=True)
    e = jnp.where(valid, jnp.exp(z - m), 0.0)
    lse = jnp.log(jnp.sum(e, axis=1, keepdims=True))
    o_out[...] = z - m - lse


def _row_spec(w):
    return pl.BlockSpec((_R, w), lambda i: (i, 0))


def _full_spec(shape):
    return pl.BlockSpec(shape, lambda i: (0,) * len(shape))


_tc1 = pl.pallas_call(
    _tc1_body,
    grid=(_N // _R,),
    in_specs=[
        _row_spec(_D), _row_spec(_D), _row_spec(16), _row_spec(16),
        _row_spec(_D),
        _full_spec((_D, _D)), _full_spec((_D, _D)), _full_spec((1, _D)),
        _full_spec((_D, _C_PAD)), _full_spec((_D, _C_PAD)),
        _full_spec((1, _C_PAD)),
    ],
    out_specs=[_row_spec(_C_PAD), _row_spec(_C_PAD)],
    out_shape=[
        jax.ShapeDtypeStruct((_N, _C_PAD), jnp.float32),
        jax.ShapeDtypeStruct((_N, _C_PAD), jnp.float32),
    ],
)

_tc2 = pl.pallas_call(
    _tc2_body,
    grid=(_N // _R,),
    in_specs=[
        _row_spec(_C_PAD), _row_spec(_C_PAD), _row_spec(16), _row_spec(16),
        _row_spec(_C_PAD),
    ],
    out_specs=[_row_spec(_C_PAD)],
    out_shape=[jax.ShapeDtypeStruct((_N, _C_PAD), jnp.float32)],
)


def kernel(x, edge_index, W1_l, W1_r, b1, W2_l, W2_r, b2):
    src = edge_index[0].astype(jnp.int32)
    dst = edge_index[1].astype(jnp.int32)

    npad = _EPAD - _E
    ar = jnp.arange(npad, dtype=jnp.int32)
    pad_src = ar % _N
    pad_dst = _N + ar % (_NROWS - _N)
    src_p = jnp.concatenate([src, pad_src]).reshape(_NW, _NCHUNK, _CH)
    dst_p = jnp.concatenate([dst, pad_dst]).reshape(_NW, _NCHUNK, _CH)

    (cnt1,) = _sc_pass0(dst_p)
    (sum1,) = _sc_pass1(x, src_p, dst_p)
    p0 = sum1[0, :_N]
    p1 = sum1[1, :_N]
    c0 = cnt1[0, :_N]
    c1 = cnt1[1, :_N]

    w2l = jnp.pad(W2_l, ((0, 0), (0, _C_PAD - _C)))
    w2r = jnp.pad(W2_r, ((0, 0), (0, _C_PAD - _C)))
    b2p = jnp.pad(b2, (0, _C_PAD - _C)).reshape(1, _C_PAD)

    g, r = _tc1(p0, p1, c0, c1, x, W1_l, W1_r, b1.reshape(1, _D), w2l, w2r, b2p)

    (sum2,) = _sc_pass2(g, src_p, dst_p)
    q0 = sum2[0, :_N]
    q1 = sum2[1, :_N]

    (o,) = _tc2(q0, q1, c0, c1, r)
    return o[:, :_C]
